# baseline (speedup 1.0000x reference)
import functools

import jax
import jax.numpy as jnp
from jax import lax
from jax.experimental import pallas as pl
from jax.experimental.pallas import tpu as pltpu
from jax.experimental.pallas import tpu_sc as plsc

SEQ = 200
EMB = 64
OUT = 2
BATCH = 4096
ROWS2 = 500000
NC, NS = 2, 16
NW = NC * NS
CPT = BATCH // NW
STRIP = 16


def _fasttext_body(text_hbm, table_hbm, w_hbm, b_hbm, out_hbm,
                   strip0_v, strip1_v, idx_cm, base_cm, rows0_v, rows1_v,
                   w_v, b_v, pooled_v, out_v,
                   ssem0, ssem1, sem0, sem1):
    wid = lax.axis_index("s") * NC + lax.axis_index("c")
    c0 = wid * CPT

    pltpu.sync_copy(w_hbm, w_v)
    pltpu.sync_copy(b_hbm, b_v.at[pl.ds(0, OUT)])

    lane0 = lax.iota(jnp.int32, 16)
    strips = (strip0_v, strip1_v)
    ssems = (ssem0, ssem1)

    def issue_strip(s, buf, sem):
        pltpu.async_copy(text_hbm.at[:, pl.ds(c0 + s * STRIP, STRIP)],
                         buf, sem)

    def drain_strip(buf, sem):
        pltpu.make_async_copy(text_hbm.at[:, pl.ds(0, STRIP)], buf,
                              sem).wait()

    issue_strip(0, strips[0], ssems[0])

    for s in range(CPT // STRIP):
        sb = s % 2
        if s + 1 < CPT // STRIP:
            issue_strip(s + 1, strips[1 - sb], ssems[1 - sb])
        drain_strip(strips[sb], ssems[sb])
        sbuf = strips[sb]
        obase = s * STRIP * SEQ

        def tr_body(i, carry, sbuf=sbuf, obase=obase):
            d = i * 16 + lane0
            cl = d // SEQ
            sq = d - cl * SEQ
            v = plsc.load_gather(sbuf, [sq, cl])
            idx_cm[pl.ds(obase + i * 16, 16)] = v >> 1
            base_cm[pl.ds(obase + i * 16, 16)] = (v & 1) * 64
            return carry

        lax.fori_loop(0, STRIP * SEQ // 16, tr_body, 0)

    inv = jnp.float32(1.0 / SEQ)
    w_regs = [[w_v[o, pl.ds(k * 16, 16)] * inv for k in range(4)]
              for o in range(2)]
    lane = lax.iota(jnp.int32, 16)
    bvec = b_v[pl.ds(0, 16)]

    zero = jnp.zeros((16,), jnp.float32)
    bufs = (rows0_v, rows1_v)
    sems = (sem0, sem1)

    def issue(c, buf, s):
        pltpu.async_copy(table_hbm.at[idx_cm.at[pl.ds(c * SEQ, 128)]],
                         buf.at[pl.ds(0, 128), :], s)
        pltpu.async_copy(table_hbm.at[idx_cm.at[pl.ds(c * SEQ + 128, SEQ - 128)]],
                         buf.at[pl.ds(128, SEQ - 128), :], s)

    def drain(buf, s):
        pltpu.make_async_copy(table_hbm.at[pl.ds(0, SEQ), :], buf, s).wait()

    def process(c, buf):
        cb = c * SEQ

        def grp_body(g, accs):
            a0, a1, a2, a3, b0, b1, b2, b3 = accs
            bases = base_cm[pl.ds(cb + g * 16, 16)]
            for j in range(0, 16, 2):
                r = g * 16 + j
                e0 = bases[j]
                e1 = bases[j + 1]
                a0 = a0 + buf[r, pl.ds(e0, 16)]
                a1 = a1 + buf[r, pl.ds(e0 + 16, 16)]
                a2 = a2 + buf[r, pl.ds(e0 + 32, 16)]
                a3 = a3 + buf[r, pl.ds(e0 + 48, 16)]
                b0 = b0 + buf[r + 1, pl.ds(e1, 16)]
                b1 = b1 + buf[r + 1, pl.ds(e1 + 16, 16)]
                b2 = b2 + buf[r + 1, pl.ds(e1 + 32, 16)]
                b3 = b3 + buf[r + 1, pl.ds(e1 + 48, 16)]
            return a0, a1, a2, a3, b0, b1, b2, b3

        accs = lax.fori_loop(0, 12, grp_body, (zero,) * 8)

        def tail_body(accs):
            a0, a1, a2, a3, b0, b1, b2, b3 = accs
            bases = base_cm[pl.ds(cb + 184, 16)]
            for j in range(8, 16, 2):
                r = 184 + j
                e0 = bases[j]
                e1 = bases[j + 1]
                a0 = a0 + buf[r, pl.ds(e0, 16)]
                a1 = a1 + buf[r, pl.ds(e0 + 16, 16)]
                a2 = a2 + buf[r, pl.ds(e0 + 32, 16)]
                a3 = a3 + buf[r, pl.ds(e0 + 48, 16)]
                b0 = b0 + buf[r + 1, pl.ds(e1, 16)]
                b1 = b1 + buf[r + 1, pl.ds(e1 + 16, 16)]
                b2 = b2 + buf[r + 1, pl.ds(e1 + 32, 16)]
                b3 = b3 + buf[r + 1, pl.ds(e1 + 48, 16)]
            return a0, a1, a2, a3, b0, b1, b2, b3

        accs = tail_body(accs)
        for k in range(4):
            pooled_v[pl.ds(c * EMB + k * 16, 16)] = accs[k] + accs[k + 4]

    issue(0, bufs[0], sems[0])

    def col_pair_body(p, carry):
        c = 2 * p
        issue(c + 1, bufs[1], sems[1])
        drain(bufs[0], sems[0])
        process(c, bufs[0])

        @pl.when(p + 1 < CPT // 2)
        def _():
            issue(c + 2, bufs[0], sems[0])

        drain(bufs[1], sems[1])
        process(c + 1, bufs[1])
        return carry

    lax.fori_loop(0, CPT // 2, col_pair_body, 0)

    for g in range(CPT // 16):
        acc0 = zero
        acc1 = zero
        base = g * 16 * EMB
        for e in range(EMB):
            eidx = lane * EMB + (base + e)
            v = plsc.load_gather(pooled_v, [eidx])
            w0e = w_regs[0][e // 16][e % 16]
            w1e = w_regs[1][e // 16][e % 16]
            acc0 = acc0 + v * w0e
            acc1 = acc1 + v * w1e
        o_base = g * 32
        plsc.store_scatter(out_v, [lane * 2 + o_base], acc0 + bvec[0])
        plsc.store_scatter(out_v, [lane * 2 + (o_base + 1)], acc1 + bvec[1])

    pltpu.sync_copy(out_v, out_hbm.at[pl.ds(wid * CPT * OUT, CPT * OUT)])


@functools.partial(
    pl.kernel,
    out_type=jax.ShapeDtypeStruct((BATCH * OUT,), jnp.float32),
    mesh=plsc.VectorSubcoreMesh(core_axis_name="c", subcore_axis_name="s",
                                num_cores=NC, num_subcores=NS),
    compiler_params=pltpu.CompilerParams(needs_layout_passes=False,
                                         use_tc_tiling_on_sc=False),
    scratch_types=[
        pltpu.VMEM((SEQ, STRIP), jnp.int32),
        pltpu.VMEM((SEQ, STRIP), jnp.int32),
        pltpu.VMEM((CPT * SEQ,), jnp.int32),
        pltpu.VMEM((CPT * SEQ,), jnp.int32),
        pltpu.VMEM((SEQ, 2 * EMB), jnp.float32),
        pltpu.VMEM((SEQ, 2 * EMB), jnp.float32),
        pltpu.VMEM((OUT, EMB), jnp.float32),
        pltpu.VMEM((16,), jnp.float32),
        pltpu.VMEM((CPT * EMB,), jnp.float32),
        pltpu.VMEM((CPT * OUT,), jnp.float32),
        pltpu.SemaphoreType.DMA,
        pltpu.SemaphoreType.DMA,
        pltpu.SemaphoreType.DMA,
        pltpu.SemaphoreType.DMA,
    ],
)
def _fasttext_sc(text, table2, w, b, out,
                 strip0_v, strip1_v, idx_cm, base_cm, rows0_v, rows1_v,
                 w_v, b_v, pooled_v, out_v, ssem0, ssem1, sem0, sem1):
    _fasttext_body(text, table2, w, b, out,
                   strip0_v, strip1_v, idx_cm, base_cm, rows0_v, rows1_v,
                   w_v, b_v, pooled_v, out_v, ssem0, ssem1, sem0, sem1)


def kernel(text, table, W, b):
    table2 = table.reshape(ROWS2, 2 * EMB)
    out_flat = _fasttext_sc(text, table2, W, b)
    return out_flat.reshape(BATCH, OUT)


# speedup vs baseline: 1.1761x; 1.1761x over previous
import functools

import jax
import jax.numpy as jnp
from jax import lax
from jax.experimental import pallas as pl
from jax.experimental.pallas import tpu as pltpu
from jax.experimental.pallas import tpu_sc as plsc

SEQ = 200
EMB = 64
OUT = 2
BATCH = 4096
NC, NS = 2, 16
NW = NC * NS
CPT = BATCH // NW
STRIP = 16
NBUF = 4


def _fasttext_body(text_hbm, table_hbm, w_hbm, b_hbm, out_hbm,
                   strip0_v, strip1_v, idx_cm,
                   rows0_v, rows1_v, rows2_v, rows3_v,
                   w_v, b_v, pooled_v, out_v,
                   ssem0, ssem1, sem0, sem1, sem2, sem3):
    wid = lax.axis_index("s") * NC + lax.axis_index("c")
    c0 = wid * CPT

    pltpu.sync_copy(w_hbm, w_v)
    pltpu.sync_copy(b_hbm, b_v.at[pl.ds(0, OUT)])

    lane0 = lax.iota(jnp.int32, 16)
    strips = (strip0_v, strip1_v)
    ssems = (ssem0, ssem1)

    def issue_strip(s, buf, sem):
        pltpu.async_copy(text_hbm.at[:, pl.ds(c0 + s * STRIP, STRIP)],
                         buf, sem)

    def drain_strip(buf, sem):
        pltpu.make_async_copy(text_hbm.at[:, pl.ds(0, STRIP)], buf,
                              sem).wait()

    issue_strip(0, strips[0], ssems[0])

    for s in range(CPT // STRIP):
        sb = s % 2
        if s + 1 < CPT // STRIP:
            issue_strip(s + 1, strips[1 - sb], ssems[1 - sb])
        drain_strip(strips[sb], ssems[sb])
        sbuf = strips[sb]
        obase = s * STRIP * SEQ

        def tr_body(i, carry, sbuf=sbuf, obase=obase):
            d = i * 16 + lane0
            cl = d // SEQ
            sq = d - cl * SEQ
            v = plsc.load_gather(sbuf, [sq, cl])
            idx_cm[pl.ds(obase + i * 16, 16)] = v
            return carry

        lax.fori_loop(0, STRIP * SEQ // 16, tr_body, 0)

    inv = jnp.float32(1.0 / SEQ)
    w_regs = [[w_v[o, pl.ds(k * 16, 16)] * inv for k in range(4)]
              for o in range(2)]
    lane = lax.iota(jnp.int32, 16)
    bvec = b_v[pl.ds(0, 16)]

    zero = jnp.zeros((16,), jnp.float32)
    bufs = (rows0_v, rows1_v, rows2_v, rows3_v)
    sems = (sem0, sem1, sem2, sem3)

    def issue(c, buf, s):
        pltpu.async_copy(table_hbm.at[idx_cm.at[pl.ds(c * SEQ, 128)]],
                         buf.at[pl.ds(0, 128), :], s)
        pltpu.async_copy(table_hbm.at[idx_cm.at[pl.ds(c * SEQ + 128, SEQ - 128)]],
                         buf.at[pl.ds(128, SEQ - 128), :], s)

    def drain(buf, s):
        pltpu.make_async_copy(table_hbm.at[pl.ds(0, SEQ), :], buf, s).wait()

    def process(c, buf):
        def grp_body(g, accs):
            a0, a1, a2, a3, b0, b1, b2, b3 = accs
            for j in range(0, 8, 2):
                r = g * 8 + j
                a0 = a0 + buf[r, pl.ds(0, 16)]
                a1 = a1 + buf[r, pl.ds(16, 16)]
                a2 = a2 + buf[r, pl.ds(32, 16)]
                a3 = a3 + buf[r, pl.ds(48, 16)]
                b0 = b0 + buf[r + 1, pl.ds(0, 16)]
                b1 = b1 + buf[r + 1, pl.ds(16, 16)]
                b2 = b2 + buf[r + 1, pl.ds(32, 16)]
                b3 = b3 + buf[r + 1, pl.ds(48, 16)]
            return a0, a1, a2, a3, b0, b1, b2, b3

        accs = lax.fori_loop(0, SEQ // 8, grp_body, (zero,) * 8)
        for k in range(4):
            pooled_v[pl.ds(c * EMB + k * 16, 16)] = accs[k] + accs[k + 4]

    issue(0, bufs[0], sems[0])
    issue(1, bufs[1], sems[1])
    issue(2, bufs[2], sems[2])

    def quad_body(q, carry):
        c = 4 * q
        issue(c + 3, bufs[3], sems[3])
        drain(bufs[0], sems[0])
        process(c, bufs[0])
        for k in range(1, 4):
            @pl.when(c + 3 + k < CPT)
            def _(k=k):
                issue(c + 3 + k, bufs[k - 1], sems[k - 1])
            drain(bufs[k], sems[k])
            process(c + k, bufs[k])
        return carry

    lax.fori_loop(0, CPT // 4, quad_body, 0)

    for g in range(CPT // 16):
        acc0 = zero
        acc1 = zero
        base = g * 16 * EMB
        for e in range(EMB):
            eidx = lane * EMB + (base + e)
            v = plsc.load_gather(pooled_v, [eidx])
            w0e = w_regs[0][e // 16][e % 16]
            w1e = w_regs[1][e // 16][e % 16]
            acc0 = acc0 + v * w0e
            acc1 = acc1 + v * w1e
        o_base = g * 32
        plsc.store_scatter(out_v, [lane * 2 + o_base], acc0 + bvec[0])
        plsc.store_scatter(out_v, [lane * 2 + (o_base + 1)], acc1 + bvec[1])

    pltpu.sync_copy(out_v, out_hbm.at[pl.ds(wid * CPT * OUT, CPT * OUT)])


@functools.partial(
    pl.kernel,
    out_type=jax.ShapeDtypeStruct((BATCH * OUT,), jnp.float32),
    mesh=plsc.VectorSubcoreMesh(core_axis_name="c", subcore_axis_name="s",
                                num_cores=NC, num_subcores=NS),
    compiler_params=pltpu.CompilerParams(needs_layout_passes=False,
                                         use_tc_tiling_on_sc=False),
    scratch_types=[
        pltpu.VMEM((SEQ, STRIP), jnp.int32),
        pltpu.VMEM((SEQ, STRIP), jnp.int32),
        pltpu.VMEM((CPT * SEQ,), jnp.int32),
        pltpu.VMEM((SEQ, EMB), jnp.float32),
        pltpu.VMEM((SEQ, EMB), jnp.float32),
        pltpu.VMEM((SEQ, EMB), jnp.float32),
        pltpu.VMEM((SEQ, EMB), jnp.float32),
        pltpu.VMEM((OUT, EMB), jnp.float32),
        pltpu.VMEM((16,), jnp.float32),
        pltpu.VMEM((CPT * EMB,), jnp.float32),
        pltpu.VMEM((CPT * OUT,), jnp.float32),
        pltpu.SemaphoreType.DMA,
        pltpu.SemaphoreType.DMA,
        pltpu.SemaphoreType.DMA,
        pltpu.SemaphoreType.DMA,
        pltpu.SemaphoreType.DMA,
        pltpu.SemaphoreType.DMA,
    ],
)
def _fasttext_sc(text, table, w, b, out,
                 strip0_v, strip1_v, idx_cm,
                 rows0_v, rows1_v, rows2_v, rows3_v,
                 w_v, b_v, pooled_v, out_v,
                 ssem0, ssem1, sem0, sem1, sem2, sem3):
    _fasttext_body(text, table, w, b, out,
                   strip0_v, strip1_v, idx_cm,
                   rows0_v, rows1_v, rows2_v, rows3_v,
                   w_v, b_v, pooled_v, out_v,
                   ssem0, ssem1, sem0, sem1, sem2, sem3)


def kernel(text, table, W, b):
    out_flat = _fasttext_sc(text, table, W, b)
    return out_flat.reshape(BATCH, OUT)
